# Initial kernel scaffold; baseline (speedup 1.0000x reference)
import jax
import jax.numpy as jnp
from jax.experimental import pallas as pl


def kernel(x, poly_coeff, reducer):
    raise NotImplementedError("write your pallas kernel here")


# speedup vs baseline: 3.8310x; 3.8310x over previous
import functools

import jax
import jax.numpy as jnp
from jax.experimental import pallas as pl
from jax.experimental.pallas import tpu as pltpu

_SRMIN = 0.0
_SRMAX = 8.0


def _embed_kernel(x_ref, idx8_ref, red_ref, tab_ref, out_ref, tile_ref, *,
                  m_count, n_coeff, w_dim, delta):
    for m in range(m_count):
        i = pl.multiple_of(idx8_ref[0, 0, m], 8) + (m % 8)
        tile_ref[pl.ds(m, 1), :] = tab_ref[pl.ds(i, 1), :]

    xr = x_ref[0][:, 0:1] - _SRMIN
    idx_f = jnp.floor(xr * (1.0 / delta))
    x0 = xr - idx_f * delta

    e = tile_ref[:, 0:w_dim]
    for i in range(1, n_coeff):
        e = e * x0 + tile_ref[:, i * w_dim:(i + 1) * w_dim]

    out_ref[0] = jnp.dot(red_ref[0], e, preferred_element_type=jnp.float32)


def kernel(x, poly_coeff, reducer):
    n_atoms, m_count, _ = x.shape
    n_grid, n_coeff, w_dim = poly_coeff.shape
    r_dim = reducer.shape[1]
    delta = (_SRMAX - _SRMIN) / n_grid

    xr = x[..., 0] - _SRMIN
    idx8 = (xr * (1.0 / delta)).astype(jnp.int32) * 8
    tab = jnp.repeat(poly_coeff.reshape(n_grid, n_coeff * w_dim), 8, axis=0)

    x3 = x.reshape(n_atoms, m_count, 1)
    idx3 = idx8.reshape(n_atoms, 1, m_count)

    grid = (n_atoms,)
    out = pl.pallas_call(
        functools.partial(_embed_kernel, m_count=m_count, n_coeff=n_coeff,
                          w_dim=w_dim, delta=delta),
        grid=grid,
        in_specs=[
            pl.BlockSpec((1, m_count, 1), lambda n: (n, 0, 0)),
            pl.BlockSpec((1, 1, m_count), lambda n: (n, 0, 0)),
            pl.BlockSpec((1, r_dim, m_count), lambda n: (n, 0, 0)),
            pl.BlockSpec((8 * n_grid, n_coeff * w_dim), lambda n: (0, 0)),
        ],
        out_specs=pl.BlockSpec((1, r_dim, w_dim), lambda n: (n, 0, 0)),
        out_shape=jax.ShapeDtypeStruct((n_atoms, r_dim, w_dim), jnp.float32),
        scratch_shapes=[pltpu.VMEM((m_count, n_coeff * w_dim), jnp.float32)],
        compiler_params=pltpu.CompilerParams(
            dimension_semantics=("parallel",),
        ),
    )(x3, idx3, reducer, tab)
    return out


# speedup vs baseline: 7.2688x; 1.8974x over previous
import functools

import jax
import jax.numpy as jnp
from jax.experimental import pallas as pl
from jax.experimental.pallas import tpu as pltpu

_SRMIN = 0.0
_SRMAX = 8.0
_BN = 8


def _embed_kernel(x_ref, idx8_ref, red_ref, tab_ref, out_ref, tile_ref, *,
                  bn, m_count, n_coeff, w_dim, delta):
    p_total = bn * m_count

    for m in range(p_total):
        i = pl.multiple_of(idx8_ref[0, 0, m], 8) + (m % 8)
        tile_ref[pl.ds(m, 1), :] = tab_ref[pl.ds(i, 1), :]

    xr = x_ref[0][:, 0:1] - _SRMIN
    idx_f = jnp.floor(xr * (1.0 / delta))
    x0 = xr - idx_f * delta

    e = tile_ref[:, 0:w_dim]
    for i in range(1, n_coeff):
        e = e * x0 + tile_ref[:, i * w_dim:(i + 1) * w_dim]

    for a in range(bn):
        out_ref[0, a] = jnp.dot(red_ref[0, a],
                                e[a * m_count:(a + 1) * m_count, :],
                                preferred_element_type=jnp.float32)


def kernel(x, poly_coeff, reducer):
    n_atoms, m_count, _ = x.shape
    n_grid, n_coeff, w_dim = poly_coeff.shape
    r_dim = reducer.shape[1]
    delta = (_SRMAX - _SRMIN) / n_grid
    bn = _BN
    n_steps = n_atoms // bn
    p_total = bn * m_count

    xr = x[..., 0] - _SRMIN
    idx8 = (xr * (1.0 / delta)).astype(jnp.int32) * 8
    tab = jnp.repeat(poly_coeff.reshape(n_grid, n_coeff * w_dim), 8, axis=0)

    x3 = x.reshape(n_steps, p_total, 1)
    idx3 = idx8.reshape(n_steps, 1, p_total)
    red3 = reducer.reshape(n_steps, bn, r_dim, m_count)

    out = pl.pallas_call(
        functools.partial(_embed_kernel, bn=bn, m_count=m_count,
                          n_coeff=n_coeff, w_dim=w_dim, delta=delta),
        grid=(n_steps,),
        in_specs=[
            pl.BlockSpec((1, p_total, 1), lambda n: (n, 0, 0)),
            pl.BlockSpec((1, 1, p_total), lambda n: (n, 0, 0),
                         memory_space=pltpu.SMEM),
            pl.BlockSpec((1, bn, r_dim, m_count), lambda n: (n, 0, 0, 0)),
            pl.BlockSpec((8 * n_grid, n_coeff * w_dim), lambda n: (0, 0)),
        ],
        out_specs=pl.BlockSpec((1, bn, r_dim, w_dim), lambda n: (n, 0, 0, 0)),
        out_shape=jax.ShapeDtypeStruct((n_steps, bn, r_dim, w_dim),
                                       jnp.float32),
        scratch_shapes=[pltpu.VMEM((p_total, n_coeff * w_dim), jnp.float32)],
        compiler_params=pltpu.CompilerParams(
            dimension_semantics=("parallel",),
        ),
    )(x3, idx3, red3, tab)
    return out.reshape(n_atoms, r_dim, w_dim)
